# baseline (speedup 1.0000x reference)
import jax
import jax.numpy as jnp
from jax import lax
from jax.experimental import pallas as pl
from jax.experimental.pallas import tpu as pltpu
from jax.experimental.pallas import tpu_sc as plsc

N = 10000
E = 320000
D = 128
NC = 2
NS = 16
NP = 10240
RPT = NP // NS
CB = 80
EPT = E // NS
NCH = EPT // CB
DH = D // 2
DQ = DH // 2
NEG = -3.0e38
PKS = 14
PKM = (1 << PKS) - 1


def _tc_head_body(fn_ref, wn_ref, wa_ref, z00_ref, z01_ref, z10_ref, z11_ref, s12_ref):
    fn = fn_ref[...]
    zb = lax.dot_general(fn, wn_ref[...], (((1,), (1,)), ((), ())),
                         preferred_element_type=jnp.float32)
    z00_ref[...] = zb[:, 0:32]
    z01_ref[...] = zb[:, 32:64]
    z10_ref[...] = zb[:, 64:96]
    z11_ref[...] = zb[:, 96:128]
    a1 = wa_ref[0, 0:128]
    a2 = wa_ref[0, 128:256]
    A = jnp.stack([a1, a2], axis=1)
    s12_ref[...] = jnp.dot(zb, A, preferred_element_type=jnp.float32)


def _tc_se_body(fe_ref, we_ref, wa_ref, se_ref):
    a3 = wa_ref[0, 256:384]
    w3 = lax.dot_general(a3, we_ref[...], (((0,), (0,)), ((), ())),
                         preferred_element_type=jnp.float32)
    prod = fe_ref[...] * w3[None, :]
    se_ref[...] = jnp.sum(prod, axis=1)[:, None]


NCH2 = NCH // 2
NST2 = NCH // 4
NPQ = NP // 4
SR = NPQ // NS


def _sc_body(pk2_hbm, s12_hbm, se_hbm, z00_hbm, z01_hbm, z10_hbm, z11_hbm,
             h_hbm,
             s12_v, src_v2, dst_v2, ex_v, m_v, mt_v,
             rows_a, rows_b,
             dn_v, red_a, red_t, shm, u_s, d_s,
             gsa, gsb, ssa, ssb, dsa, dsb):
    c = lax.axis_index("c")
    s = lax.axis_index("s")
    pltpu.sync_copy(pk2_hbm.at[s], src_v2)
    pltpu.sync_copy(s12_hbm.at[0], s12_v.at[pl.ds(0, N)])
    pltpu.sync_copy(s12_hbm.at[1], s12_v.at[pl.ds(N, N)])
    pltpu.sync_copy(se_hbm.at[s], ex_v)

    def unpack(i, carry):
        for k in range(CB // 16):
            slk = pl.ds(k * 16, 16)
            pk = src_v2[i, slk]
            dst_v2[i, slk] = lax.shift_right_logical(pk, PKS)
            src_v2[i, slk] = jnp.bitwise_and(pk, PKM)
        return carry

    lax.fori_loop(0, NCH, unpack, 0)

    neg = jnp.full((16,), NEG, jnp.float32)

    def init_m(k, carry):
        m_v[pl.ds(k * 16, 16)] = neg
        return carry

    lax.fori_loop(0, NP // 16, init_m, 0)

    def edge_step(i, carry):
        for k in range(CB // 16):
            slk = pl.ds(k * 16, 16)
            sl = pl.ds(i * CB + k * 16, 16)
            srcv = src_v2[i, slk]
            dstv = dst_v2[i, slk]
            ev = (plsc.load_gather(s12_v, [srcv])
                  + plsc.load_gather(s12_v, [dstv + N]) + ex_v[sl])
            ev = jnp.where(ev > 0, ev, 0.2 * ev)
            ex_v[sl] = ev

            def upd_body(go):
                cur = plsc.load_gather(m_v, [dstv])
                need = ev > cur
                plsc.store_scatter(m_v, [dstv], ev, mask=need)
                cur2 = plsc.load_gather(m_v, [dstv])
                return jnp.any(ev > cur2)

            lax.while_loop(lambda go: go, upd_body,
                           jnp.any(ev > plsc.load_gather(m_v, [dstv])))
        return carry

    lax.fori_loop(0, NCH, edge_step, 0)

    def phase(p, carry):
        pltpu.sync_copy(m_v.at[pl.ds(p * NPQ, NPQ)], shm.at[s])
        plsc.subcore_barrier()
        rb = s * SR

        def jred(j, carry2):
            pltpu.sync_copy(shm.at[j, pl.ds(rb, SR)], red_t)

            def red(k, carry3):
                sl = pl.ds(k * 16, 16)
                red_a[sl] = jnp.maximum(red_a[sl], red_t[sl])
                return carry3

            lax.fori_loop(0, SR // 16, red, 0)
            return carry2

        pltpu.sync_copy(shm.at[0, pl.ds(rb, SR)], red_a)
        lax.fori_loop(1, NS, jred, 0)
        pltpu.sync_copy(red_a, shm.at[0, pl.ds(rb, SR)])
        plsc.subcore_barrier()
        pltpu.sync_copy(shm.at[0], m_v.at[pl.ds(p * NPQ, NPQ)])
        plsc.subcore_barrier()
        return carry

    lax.fori_loop(0, NP // NPQ, phase, 0)

    zro = jnp.zeros((16,), jnp.float32)

    def zb_flat(k, carry):
        mt_v[pl.ds(k * 16, 16)] = zro
        return carry

    lax.fori_loop(0, NP // 16, zb_flat, 0)

    @pl.when(s == 0)
    def _():
        pltpu.sync_copy(mt_v.at[pl.ds(0, N)], d_s)

    def exf(i, carry):
        for k in range(CB // 16):
            slk = pl.ds(k * 16, 16)
            dstv = dst_v2[i, slk]
            sl = pl.ds(i * CB + k * 16, 16)
            mg = plsc.load_gather(m_v, [dstv])
            ex_v[sl] = jnp.exp(ex_v[sl] - mg)
        return carry

    lax.fori_loop(0, NCH, exf, 0)

    rbase = s * RPT

    def gwait(buf, sem):
        pltpu.make_async_copy(z00_hbm.at[src_v2.at[0]], buf, sem).wait()

    def swait(buf, sem):
        pltpu.make_async_copy(buf, u_s.at[dst_v2.at[0]], sem).wait()

    def dwait(sem):
        pltpu.make_async_copy(ex_v.at[pl.ds(0, CB)], d_s.at[dst_v2.at[0]],
                              sem).wait()

    def scale(i, buf, lo=0, hi=CB // 4):
        ebase = i * CB

        def sbody(j2, carry):
            for jj in range(4):
                j = j2 * 4 + jj
                spl = plsc.load_gather(
                    ex_v, [jnp.full((16,), ebase + j, jnp.int32)])
                for col in range(DQ // 16):
                    slc = pl.ds(col * 16, 16)
                    buf[j, slc] = buf[j, slc] * spl
            return carry

        lax.fori_loop(lo, hi, sbody, 0)

    for q in range(2):
        zc0_hbm = z00_hbm if q == 0 else z01_hbm
        zc1_hbm = z10_hbm if q == 0 else z11_hbm

        def gstart(i, buf, sem):
            @pl.when(c == 0)
            def _():
                pltpu.async_copy(zc0_hbm.at[src_v2.at[i]], buf, sem)

            @pl.when(c == 1)
            def _():
                pltpu.async_copy(zc1_hbm.at[src_v2.at[i]], buf, sem)

        def sstart(i, buf, sem, dsem):
            pltpu.async_copy(buf, u_s.at[dst_v2.at[i]], sem, add=True)
            if q == 0:
                pltpu.async_copy(ex_v.at[pl.ds(i * CB, CB)],
                                 d_s.at[dst_v2.at[i]], dsem, add=True)

        def sdone(buf, sem, dsem):
            swait(buf, sem)
            if q == 0:
                dwait(dsem)

        def zrow(j, carry):
            for k in range(DQ // 16):
                rows_a[j, pl.ds(k * 16, 16)] = zro
            return carry

        lax.fori_loop(0, CB, zrow, 0)

        def zcopy(t, carry):
            row0 = rbase + t * CB

            @pl.when(row0 < N)
            def _():
                pltpu.sync_copy(rows_a, u_s.at[pl.ds(row0, CB), :])
            return carry

        lax.fori_loop(0, RPT // CB, zcopy, 0)
        plsc.subcore_barrier()

        gstart(0, rows_a, gsa)

        def pipe(t, carry):
            i0 = 2 * t
            i1 = 2 * t + 1

            @pl.when(t > 0)
            def _():
                sdone(rows_b, ssb, dsb)

            gstart(i1, rows_b, gsb)
            gwait(rows_a, gsa)
            scale(i0, rows_a)
            sstart(i0, rows_a, ssa, dsa)

            gwait(rows_b, gsb)
            scale(i1, rows_b, 0, CB // 8)

            @pl.when(t + 1 < NCH2)
            def _():
                sdone(rows_a, ssa, dsa)
                gstart(i0 + 2, rows_a, gsa)

            scale(i1, rows_b, CB // 8, CB // 4)
            sstart(i1, rows_b, ssb, dsb)

            return carry

        lax.fori_loop(0, NCH2, pipe, 0)
        sdone(rows_a, ssa, dsa)
        sdone(rows_b, ssb, dsb)

        plsc.subcore_barrier()

        def dump(t, carry):
            row0 = rbase + t * CB

            @pl.when(row0 < N)
            def _():
                pltpu.sync_copy(u_s.at[pl.ds(row0, CB), :], rows_a)
                pltpu.sync_copy(d_s.at[pl.ds(row0, CB)], dn_v)
                for k in range(CB // 16):
                    slk = pl.ds(k * 16, 16)
                    dn_v[slk] = 1.0 / (dn_v[slk] + 1e-9)

                def div(j2, carry):
                    for jj in range(4):
                        j = j2 * 4 + jj
                        spl = plsc.load_gather(
                            dn_v, [jnp.full((16,), j, jnp.int32)])
                        for col in range(DQ // 16):
                            slc = pl.ds(col * 16, 16)
                            rows_a[j, slc] = rows_a[j, slc] * spl
                    return carry

                lax.fori_loop(0, CB // 4, div, 0)
                pltpu.sync_copy(rows_a, h_hbm.at[c, q, pl.ds(row0, CB), :])
            return carry

        lax.fori_loop(0, RPT // CB, dump, 0)
        plsc.subcore_barrier()


def kernel(feats_node, feats_edge, edge_index, W_node, W_edge, W_attn):
    src_i = edge_index[0].astype(jnp.int32)
    dst_i = edge_index[1].astype(jnp.int32)
    packed = jnp.left_shift(dst_i, PKS) + src_i

    BN = 1000
    z00, z01, z10, z11, s12 = pl.pallas_call(
        _tc_head_body,
        grid=(N // BN,),
        in_specs=[
            pl.BlockSpec((BN, 128), lambda i: (i, 0)),
            pl.BlockSpec((128, 128), lambda i: (0, 0)),
            pl.BlockSpec((1, 384), lambda i: (0, 0)),
        ],
        out_specs=[
            pl.BlockSpec((BN, 32), lambda i: (i, 0)),
            pl.BlockSpec((BN, 32), lambda i: (i, 0)),
            pl.BlockSpec((BN, 32), lambda i: (i, 0)),
            pl.BlockSpec((BN, 32), lambda i: (i, 0)),
            pl.BlockSpec((BN, 2), lambda i: (i, 0)),
        ],
        out_shape=[
            jax.ShapeDtypeStruct((N, 32), jnp.float32),
            jax.ShapeDtypeStruct((N, 32), jnp.float32),
            jax.ShapeDtypeStruct((N, 32), jnp.float32),
            jax.ShapeDtypeStruct((N, 32), jnp.float32),
            jax.ShapeDtypeStruct((N, 2), jnp.float32),
        ],
    )(feats_node, W_node, W_attn)

    BE = 3200
    se = pl.pallas_call(
        _tc_se_body,
        grid=(E // BE,),
        in_specs=[
            pl.BlockSpec((BE, 16), lambda i: (i, 0)),
            pl.BlockSpec((128, 16), lambda i: (0, 0)),
            pl.BlockSpec((1, 384), lambda i: (0, 0)),
        ],
        out_specs=pl.BlockSpec((BE, 1), lambda i: (i, 0)),
        out_shape=jax.ShapeDtypeStruct((E, 1), jnp.float32),
    )(feats_edge, W_edge, W_attn).reshape(NS, EPT)
    s12t = s12.T
    pk2 = packed.reshape(NS, NCH, CB)

    mesh = plsc.VectorSubcoreMesh(core_axis_name="c", subcore_axis_name="s")

    h2 = pl.kernel(
        _sc_body,
        out_type=jax.ShapeDtypeStruct((NC, 2, N, DQ), jnp.float32),
        mesh=mesh,
        compiler_params=pltpu.CompilerParams(needs_layout_passes=False,
                                             use_tc_tiling_on_sc=False),
        scratch_types=[
            pltpu.VMEM((2 * N,), jnp.float32),
            pltpu.VMEM((NCH, CB), jnp.int32),
            pltpu.VMEM((NCH, CB), jnp.int32),
            pltpu.VMEM((EPT,), jnp.float32),
            pltpu.VMEM((NP,), jnp.float32),
            pltpu.VMEM((NP,), jnp.float32),
            pltpu.VMEM((CB, DQ), jnp.float32),
            pltpu.VMEM((CB, DQ), jnp.float32),
            pltpu.VMEM((CB,), jnp.float32),
            pltpu.VMEM((SR,), jnp.float32),
            pltpu.VMEM((SR,), jnp.float32),
            pltpu.VMEM_SHARED((NS, NPQ), jnp.float32),
            pltpu.VMEM_SHARED((N, DQ), jnp.float32),
            pltpu.VMEM_SHARED((N,), jnp.float32),
            pltpu.SemaphoreType.DMA,
            pltpu.SemaphoreType.DMA,
            pltpu.SemaphoreType.DMA,
            pltpu.SemaphoreType.DMA,
            pltpu.SemaphoreType.DMA,
            pltpu.SemaphoreType.DMA,
        ],
    )(pk2, s12t, se, z00, z01, z10, z11)
    return jnp.moveaxis(h2.reshape(4, N, DQ), 0, 1).reshape(N, D)


# speedup vs baseline: 1.0146x; 1.0146x over previous
import jax
import jax.numpy as jnp
from jax import lax
from jax.experimental import pallas as pl
from jax.experimental.pallas import tpu as pltpu
from jax.experimental.pallas import tpu_sc as plsc

N = 10000
E = 320000
D = 128
NC = 2
NS = 16
NP = 10240
RPT = NP // NS
CB = 80
EPT = E // NS
NCH = EPT // CB
DH = D // 2
DQ = DH // 2
NEG = -3.0e38
PKS = 14
PKM = (1 << PKS) - 1


def _tc_head_body(fn_ref, wn_ref, wa_ref, z00_ref, z01_ref, z10_ref, z11_ref, s12_ref):
    fn = fn_ref[...]
    zb = lax.dot_general(fn, wn_ref[...], (((1,), (1,)), ((), ())),
                         preferred_element_type=jnp.float32)
    z00_ref[...] = zb[:, 0:32]
    z01_ref[...] = zb[:, 32:64]
    z10_ref[...] = zb[:, 64:96]
    z11_ref[...] = zb[:, 96:128]
    a1 = wa_ref[0, 0:128]
    a2 = wa_ref[0, 128:256]
    A = jnp.stack([a1, a2], axis=1)
    s12_ref[...] = jnp.dot(zb, A, preferred_element_type=jnp.float32)


def _tc_se_body(fe_ref, we_ref, wa_ref, se_ref):
    a3 = wa_ref[0, 256:384]
    w3 = lax.dot_general(a3, we_ref[...], (((0,), (0,)), ((), ())),
                         preferred_element_type=jnp.float32)
    prod = fe_ref[...] * w3[None, :]
    se_ref[...] = jnp.sum(prod, axis=1)[:, None]


NCH2 = NCH // 2
NST2 = NCH // 4
NPQ = NP // 4
SR = NPQ // NS


def _sc_body(pk2_hbm, s12_hbm, se_hbm, z00_hbm, z01_hbm, z10_hbm, z11_hbm,
             h_hbm,
             s12_v, src_v2, dst_v2, ex_v, m_v, mt_v,
             rows_a, rows_b,
             dn_v, red_a, red_t, shm, u_s, d_s,
             gsa, gsb, ssa, ssb, dsa, dsb):
    c = lax.axis_index("c")
    s = lax.axis_index("s")
    pltpu.sync_copy(pk2_hbm.at[s], src_v2)
    pltpu.sync_copy(s12_hbm.at[0], s12_v.at[pl.ds(0, N)])
    pltpu.sync_copy(s12_hbm.at[1], s12_v.at[pl.ds(N, N)])
    pltpu.sync_copy(se_hbm.at[s], ex_v)

    def unpack(i, carry):
        for k in range(CB // 16):
            slk = pl.ds(k * 16, 16)
            pk = src_v2[i, slk]
            dst_v2[i, slk] = lax.shift_right_logical(pk, PKS)
            src_v2[i, slk] = jnp.bitwise_and(pk, PKM)
        return carry

    lax.fori_loop(0, NCH, unpack, 0)

    neg = jnp.full((16,), NEG, jnp.float32)

    def init_m(k, carry):
        m_v[pl.ds(k * 16, 16)] = neg
        return carry

    lax.fori_loop(0, NP // 16, init_m, 0)

    def edge_step(i, carry):
        for k in range(CB // 16):
            slk = pl.ds(k * 16, 16)
            sl = pl.ds(i * CB + k * 16, 16)
            srcv = src_v2[i, slk]
            dstv = dst_v2[i, slk]
            ev = (plsc.load_gather(s12_v, [srcv])
                  + plsc.load_gather(s12_v, [dstv + N]) + ex_v[sl])
            ev = jnp.where(ev > 0, ev, 0.2 * ev)
            ex_v[sl] = ev

            def upd_body(go):
                cur = plsc.load_gather(m_v, [dstv])
                need = ev > cur
                plsc.store_scatter(m_v, [dstv], ev, mask=need)
                cur2 = plsc.load_gather(m_v, [dstv])
                return jnp.any(ev > cur2)

            lax.while_loop(lambda go: go, upd_body,
                           jnp.any(ev > plsc.load_gather(m_v, [dstv])))
        return carry

    lax.fori_loop(0, NCH, edge_step, 0)

    def phase(p, carry):
        pltpu.sync_copy(m_v.at[pl.ds(p * NPQ, NPQ)], shm.at[s])
        plsc.subcore_barrier()
        rb = s * SR

        def jred(j, carry2):
            pltpu.sync_copy(shm.at[j, pl.ds(rb, SR)], red_t)

            def red(k, carry3):
                sl = pl.ds(k * 16, 16)
                red_a[sl] = jnp.maximum(red_a[sl], red_t[sl])
                return carry3

            lax.fori_loop(0, SR // 16, red, 0)
            return carry2

        pltpu.sync_copy(shm.at[0, pl.ds(rb, SR)], red_a)
        lax.fori_loop(1, NS, jred, 0)
        pltpu.sync_copy(red_a, shm.at[0, pl.ds(rb, SR)])
        plsc.subcore_barrier()
        pltpu.sync_copy(shm.at[0], m_v.at[pl.ds(p * NPQ, NPQ)])
        plsc.subcore_barrier()
        return carry

    lax.fori_loop(0, NP // NPQ, phase, 0)

    zro = jnp.zeros((16,), jnp.float32)

    def zb_flat(k, carry):
        mt_v[pl.ds(k * 16, 16)] = zro
        return carry

    lax.fori_loop(0, NP // 16, zb_flat, 0)

    @pl.when(s == 0)
    def _():
        pltpu.sync_copy(mt_v.at[pl.ds(0, N)], d_s)

    def exf(i, carry):
        for k in range(CB // 16):
            slk = pl.ds(k * 16, 16)
            dstv = dst_v2[i, slk]
            sl = pl.ds(i * CB + k * 16, 16)
            mg = plsc.load_gather(m_v, [dstv])
            ex_v[sl] = jnp.exp(ex_v[sl] - mg)
        return carry

    lax.fori_loop(0, NCH, exf, 0)

    rbase = s * RPT

    def gwait(buf, sem):
        pltpu.make_async_copy(z00_hbm.at[src_v2.at[0]], buf, sem).wait()

    def swait(buf, sem):
        pltpu.make_async_copy(buf, u_s.at[dst_v2.at[0]], sem).wait()

    def dwait(sem):
        pltpu.make_async_copy(ex_v.at[pl.ds(0, CB)], d_s.at[dst_v2.at[0]],
                              sem).wait()

    def scale(i, buf):
        ebase = i * CB

        def sbody(j2, carry):
            for jj in range(8):
                j = j2 * 8 + jj
                spl = plsc.load_gather(
                    ex_v, [jnp.full((16,), ebase + j, jnp.int32)])
                for col in range(DQ // 16):
                    slc = pl.ds(col * 16, 16)
                    buf[j, slc] = buf[j, slc] * spl
            return carry

        lax.fori_loop(0, CB // 8, sbody, 0)

    for q in range(2):
        zc0_hbm = z00_hbm if q == 0 else z01_hbm
        zc1_hbm = z10_hbm if q == 0 else z11_hbm

        def gstart(i, buf, sem):
            @pl.when(c == 0)
            def _():
                pltpu.async_copy(zc0_hbm.at[src_v2.at[i]], buf, sem)

            @pl.when(c == 1)
            def _():
                pltpu.async_copy(zc1_hbm.at[src_v2.at[i]], buf, sem)

        def sstart(i, buf, sem, dsem):
            pltpu.async_copy(buf, u_s.at[dst_v2.at[i]], sem, add=True)
            if q == 0:
                pltpu.async_copy(ex_v.at[pl.ds(i * CB, CB)],
                                 d_s.at[dst_v2.at[i]], dsem, add=True)

        def sdone(buf, sem, dsem):
            swait(buf, sem)
            if q == 0:
                dwait(dsem)

        def zrow(j, carry):
            for k in range(DQ // 16):
                rows_a[j, pl.ds(k * 16, 16)] = zro
            return carry

        lax.fori_loop(0, CB, zrow, 0)

        def zcopy(t, carry):
            row0 = rbase + t * CB

            @pl.when(row0 < N)
            def _():
                pltpu.sync_copy(rows_a, u_s.at[pl.ds(row0, CB), :])
            return carry

        lax.fori_loop(0, RPT // CB, zcopy, 0)
        plsc.subcore_barrier()

        gstart(0, rows_a, gsa)

        def pipe(t, carry):
            i0 = 2 * t
            i1 = 2 * t + 1

            @pl.when(t > 0)
            def _():
                sdone(rows_b, ssb, dsb)

            gstart(i1, rows_b, gsb)
            gwait(rows_a, gsa)
            scale(i0, rows_a)
            sstart(i0, rows_a, ssa, dsa)

            @pl.when(t + 1 < NCH2)
            def _():
                sdone(rows_a, ssa, dsa)
                gstart(i0 + 2, rows_a, gsa)

            gwait(rows_b, gsb)
            scale(i1, rows_b)
            sstart(i1, rows_b, ssb, dsb)

            return carry

        lax.fori_loop(0, NCH2, pipe, 0)
        sdone(rows_a, ssa, dsa)
        sdone(rows_b, ssb, dsb)

        plsc.subcore_barrier()

        def dump(t, carry):
            row0 = rbase + t * CB

            @pl.when(row0 < N)
            def _():
                pltpu.sync_copy(u_s.at[pl.ds(row0, CB), :], rows_a)
                pltpu.sync_copy(d_s.at[pl.ds(row0, CB)], dn_v)
                for k in range(CB // 16):
                    slk = pl.ds(k * 16, 16)
                    dn_v[slk] = 1.0 / (dn_v[slk] + 1e-9)

                def div(j2, carry):
                    for jj in range(4):
                        j = j2 * 4 + jj
                        spl = plsc.load_gather(
                            dn_v, [jnp.full((16,), j, jnp.int32)])
                        for col in range(DQ // 16):
                            slc = pl.ds(col * 16, 16)
                            rows_a[j, slc] = rows_a[j, slc] * spl
                    return carry

                lax.fori_loop(0, CB // 4, div, 0)
                pltpu.sync_copy(rows_a, h_hbm.at[c, q, pl.ds(row0, CB), :])
            return carry

        lax.fori_loop(0, RPT // CB, dump, 0)
        plsc.subcore_barrier()


def kernel(feats_node, feats_edge, edge_index, W_node, W_edge, W_attn):
    src_i = edge_index[0].astype(jnp.int32)
    dst_i = edge_index[1].astype(jnp.int32)
    packed = jnp.left_shift(dst_i, PKS) + src_i

    BN = 1000
    z00, z01, z10, z11, s12 = pl.pallas_call(
        _tc_head_body,
        grid=(N // BN,),
        in_specs=[
            pl.BlockSpec((BN, 128), lambda i: (i, 0)),
            pl.BlockSpec((128, 128), lambda i: (0, 0)),
            pl.BlockSpec((1, 384), lambda i: (0, 0)),
        ],
        out_specs=[
            pl.BlockSpec((BN, 32), lambda i: (i, 0)),
            pl.BlockSpec((BN, 32), lambda i: (i, 0)),
            pl.BlockSpec((BN, 32), lambda i: (i, 0)),
            pl.BlockSpec((BN, 32), lambda i: (i, 0)),
            pl.BlockSpec((BN, 2), lambda i: (i, 0)),
        ],
        out_shape=[
            jax.ShapeDtypeStruct((N, 32), jnp.float32),
            jax.ShapeDtypeStruct((N, 32), jnp.float32),
            jax.ShapeDtypeStruct((N, 32), jnp.float32),
            jax.ShapeDtypeStruct((N, 32), jnp.float32),
            jax.ShapeDtypeStruct((N, 2), jnp.float32),
        ],
    )(feats_node, W_node, W_attn)

    BE = 3200
    se = pl.pallas_call(
        _tc_se_body,
        grid=(E // BE,),
        in_specs=[
            pl.BlockSpec((BE, 16), lambda i: (i, 0)),
            pl.BlockSpec((128, 16), lambda i: (0, 0)),
            pl.BlockSpec((1, 384), lambda i: (0, 0)),
        ],
        out_specs=pl.BlockSpec((BE, 1), lambda i: (i, 0)),
        out_shape=jax.ShapeDtypeStruct((E, 1), jnp.float32),
    )(feats_edge, W_edge, W_attn).reshape(NS, EPT)
    s12t = s12.T
    pk2 = packed.reshape(NS, NCH, CB)

    mesh = plsc.VectorSubcoreMesh(core_axis_name="c", subcore_axis_name="s")

    h2 = pl.kernel(
        _sc_body,
        out_type=jax.ShapeDtypeStruct((NC, 2, N, DQ), jnp.float32),
        mesh=mesh,
        compiler_params=pltpu.CompilerParams(needs_layout_passes=False,
                                             use_tc_tiling_on_sc=False),
        scratch_types=[
            pltpu.VMEM((2 * N,), jnp.float32),
            pltpu.VMEM((NCH, CB), jnp.int32),
            pltpu.VMEM((NCH, CB), jnp.int32),
            pltpu.VMEM((EPT,), jnp.float32),
            pltpu.VMEM((NP,), jnp.float32),
            pltpu.VMEM((NP,), jnp.float32),
            pltpu.VMEM((CB, DQ), jnp.float32),
            pltpu.VMEM((CB, DQ), jnp.float32),
            pltpu.VMEM((CB,), jnp.float32),
            pltpu.VMEM((SR,), jnp.float32),
            pltpu.VMEM((SR,), jnp.float32),
            pltpu.VMEM_SHARED((NS, NPQ), jnp.float32),
            pltpu.VMEM_SHARED((N, DQ), jnp.float32),
            pltpu.VMEM_SHARED((N,), jnp.float32),
            pltpu.SemaphoreType.DMA,
            pltpu.SemaphoreType.DMA,
            pltpu.SemaphoreType.DMA,
            pltpu.SemaphoreType.DMA,
            pltpu.SemaphoreType.DMA,
            pltpu.SemaphoreType.DMA,
        ],
    )(pk2, s12t, se, z00, z01, z10, z11)
    return jnp.moveaxis(h2.reshape(4, N, DQ), 0, 1).reshape(N, D)


# speedup vs baseline: 1.0165x; 1.0019x over previous
import jax
import jax.numpy as jnp
from jax import lax
from jax.experimental import pallas as pl
from jax.experimental.pallas import tpu as pltpu
from jax.experimental.pallas import tpu_sc as plsc

N = 10000
E = 320000
D = 128
NC = 2
NS = 16
NP = 10240
RPT = NP // NS
CB = 80
EPT = E // NS
NCH = EPT // CB
DH = D // 2
DQ = DH // 2
NEG = -3.0e38
PKS = 14
PKM = (1 << PKS) - 1


def _tc_head_body(fn_ref, wn_ref, wa_ref, z00_ref, z01_ref, z10_ref, z11_ref, s12_ref):
    fn = fn_ref[...]
    zb = lax.dot_general(fn, wn_ref[...], (((1,), (1,)), ((), ())),
                         preferred_element_type=jnp.float32)
    z00_ref[...] = zb[:, 0:32]
    z01_ref[...] = zb[:, 32:64]
    z10_ref[...] = zb[:, 64:96]
    z11_ref[...] = zb[:, 96:128]
    a1 = wa_ref[0, 0:128]
    a2 = wa_ref[0, 128:256]
    A = jnp.stack([a1, a2], axis=1)
    s12_ref[...] = jnp.dot(zb, A, preferred_element_type=jnp.float32)


def _tc_se_body(fe_ref, we_ref, wa_ref, se_ref):
    a3 = wa_ref[0, 256:384]
    w3 = lax.dot_general(a3, we_ref[...], (((0,), (0,)), ((), ())),
                         preferred_element_type=jnp.float32)
    prod = fe_ref[...] * w3[None, :]
    se_ref[...] = jnp.sum(prod, axis=1)[:, None]


NCH2 = NCH // 2
NST2 = NCH // 4
NPQ = NP // 4
SR = NPQ // NS


def _sc_body(pk2_hbm, s12_hbm, se_hbm, z00_hbm, z01_hbm, z10_hbm, z11_hbm,
             h_hbm,
             s12_v, src_v2, dst_v2, ex_v, m_v, mt_v,
             rows_a, rows_b,
             dn_v, red_a, red_t, shm, u_s, d_s,
             gsa, gsb, ssa, ssb, dsa, dsb):
    c = lax.axis_index("c")
    s = lax.axis_index("s")
    pltpu.sync_copy(pk2_hbm.at[s], src_v2)
    pltpu.sync_copy(s12_hbm.at[0], s12_v.at[pl.ds(0, N)])
    pltpu.sync_copy(s12_hbm.at[1], s12_v.at[pl.ds(N, N)])
    pltpu.sync_copy(se_hbm.at[s], ex_v)

    def unpack(i, carry):
        for k in range(CB // 16):
            slk = pl.ds(k * 16, 16)
            pk = src_v2[i, slk]
            dst_v2[i, slk] = lax.shift_right_logical(pk, PKS)
            src_v2[i, slk] = jnp.bitwise_and(pk, PKM)
        return carry

    lax.fori_loop(0, NCH, unpack, 0)

    neg = jnp.full((16,), NEG, jnp.float32)

    def init_m(k, carry):
        m_v[pl.ds(k * 16, 16)] = neg
        return carry

    lax.fori_loop(0, NP // 16, init_m, 0)

    def edge_step(i, carry):
        for k in range(CB // 16):
            slk = pl.ds(k * 16, 16)
            sl = pl.ds(i * CB + k * 16, 16)
            srcv = src_v2[i, slk]
            dstv = dst_v2[i, slk]
            ev = (plsc.load_gather(s12_v, [srcv])
                  + plsc.load_gather(s12_v, [dstv + N]) + ex_v[sl])
            ev = jnp.where(ev > 0, ev, 0.2 * ev)
            ex_v[sl] = ev

            def upd_body(go):
                cur = plsc.load_gather(m_v, [dstv])
                need = ev > cur
                plsc.store_scatter(m_v, [dstv], ev, mask=need)
                cur2 = plsc.load_gather(m_v, [dstv])
                return jnp.any(ev > cur2)

            lax.while_loop(lambda go: go, upd_body,
                           jnp.any(ev > plsc.load_gather(m_v, [dstv])))
        return carry

    lax.fori_loop(0, NCH, edge_step, 0)

    def phase(p, carry):
        pltpu.sync_copy(m_v.at[pl.ds(p * NPQ, NPQ)], shm.at[s])
        plsc.subcore_barrier()
        rb = s * SR

        def jred(j, carry2):
            pltpu.sync_copy(shm.at[j, pl.ds(rb, SR)], red_t)

            def red(k, carry3):
                sl = pl.ds(k * 16, 16)
                red_a[sl] = jnp.maximum(red_a[sl], red_t[sl])
                return carry3

            lax.fori_loop(0, SR // 16, red, 0)
            return carry2

        pltpu.sync_copy(shm.at[0, pl.ds(rb, SR)], red_a)
        lax.fori_loop(1, NS, jred, 0)
        pltpu.sync_copy(red_a, shm.at[0, pl.ds(rb, SR)])
        plsc.subcore_barrier()
        pltpu.sync_copy(shm.at[0], m_v.at[pl.ds(p * NPQ, NPQ)])
        plsc.subcore_barrier()
        return carry

    lax.fori_loop(0, NP // NPQ, phase, 0)

    zro = jnp.zeros((16,), jnp.float32)

    def zb_flat(k, carry):
        mt_v[pl.ds(k * 16, 16)] = zro
        return carry

    lax.fori_loop(0, NP // 16, zb_flat, 0)

    @pl.when(s == 0)
    def _():
        pltpu.sync_copy(mt_v.at[pl.ds(0, N)], d_s)

    def exf(i, carry):
        for k in range(CB // 16):
            slk = pl.ds(k * 16, 16)
            dstv = dst_v2[i, slk]
            sl = pl.ds(i * CB + k * 16, 16)
            mg = plsc.load_gather(m_v, [dstv])
            ex_v[sl] = jnp.exp(ex_v[sl] - mg)
        return carry

    lax.fori_loop(0, NCH, exf, 0)

    rbase = s * RPT

    def gwait(buf, sem):
        pltpu.make_async_copy(z00_hbm.at[src_v2.at[0]], buf, sem).wait()

    def swait(buf, sem):
        pltpu.make_async_copy(buf, u_s.at[dst_v2.at[0]], sem).wait()

    def dwait(sem):
        pltpu.make_async_copy(ex_v.at[pl.ds(0, CB)], d_s.at[dst_v2.at[0]],
                              sem).wait()

    def scale(i, buf):
        ebase = i * CB

        def sbody(j2, carry):
            for jj in range(4):
                j = j2 * 4 + jj
                spl = plsc.load_gather(
                    ex_v, [jnp.full((16,), ebase + j, jnp.int32)])
                for col in range(DQ // 16):
                    slc = pl.ds(col * 16, 16)
                    buf[j, slc] = buf[j, slc] * spl
            return carry

        lax.fori_loop(0, CB // 4, sbody, 0)

    for q in range(2):
        zc0_hbm = z00_hbm if q == 0 else z01_hbm
        zc1_hbm = z10_hbm if q == 0 else z11_hbm

        def gstart(i, buf, sem):
            @pl.when(c == 0)
            def _():
                pltpu.async_copy(zc0_hbm.at[src_v2.at[i]], buf, sem)

            @pl.when(c == 1)
            def _():
                pltpu.async_copy(zc1_hbm.at[src_v2.at[i]], buf, sem)

        def sstart(i, buf, sem, dsem):
            pltpu.async_copy(buf, u_s.at[dst_v2.at[i]], sem, add=True)
            if q == 0:
                pltpu.async_copy(ex_v.at[pl.ds(i * CB, CB)],
                                 d_s.at[dst_v2.at[i]], dsem, add=True)

        def sdone(buf, sem, dsem):
            swait(buf, sem)
            if q == 0:
                dwait(dsem)

        def zrow(j, carry):
            for k in range(DQ // 16):
                rows_a[j, pl.ds(k * 16, 16)] = zro
            return carry

        lax.fori_loop(0, CB, zrow, 0)

        def zcopy(t, carry):
            row0 = rbase + t * CB

            @pl.when(row0 < N)
            def _():
                pltpu.sync_copy(rows_a, u_s.at[pl.ds(row0, CB), :])
            return carry

        lax.fori_loop(0, RPT // CB, zcopy, 0)
        plsc.subcore_barrier()

        gstart(0, rows_a, gsa)

        def pipe(t, carry):
            i0 = 2 * t
            i1 = 2 * t + 1

            @pl.when(t > 0)
            def _():
                sdone(rows_b, ssb, dsb)

            gstart(i1, rows_b, gsb)
            gwait(rows_a, gsa)
            scale(i0, rows_a)
            sstart(i0, rows_a, ssa, dsa)

            @pl.when(t + 1 < NCH2)
            def _():
                sdone(rows_a, ssa, dsa)
                gstart(i0 + 2, rows_a, gsa)

            gwait(rows_b, gsb)
            scale(i1, rows_b)
            sstart(i1, rows_b, ssb, dsb)

            return carry

        lax.fori_loop(0, NCH2, pipe, 0)
        sdone(rows_a, ssa, dsa)
        sdone(rows_b, ssb, dsb)

        plsc.subcore_barrier()

        def dump(t, carry):
            row0 = rbase + t * CB

            @pl.when(row0 < N)
            def _():
                pltpu.sync_copy(u_s.at[pl.ds(row0, CB), :], rows_a)
                pltpu.sync_copy(d_s.at[pl.ds(row0, CB)], dn_v)
                for k in range(CB // 16):
                    slk = pl.ds(k * 16, 16)
                    dn_v[slk] = 1.0 / (dn_v[slk] + 1e-9)

                def div(j2, carry):
                    for jj in range(4):
                        j = j2 * 4 + jj
                        spl = plsc.load_gather(
                            dn_v, [jnp.full((16,), j, jnp.int32)])
                        for col in range(DQ // 16):
                            slc = pl.ds(col * 16, 16)
                            rows_a[j, slc] = rows_a[j, slc] * spl
                    return carry

                lax.fori_loop(0, CB // 4, div, 0)
                pltpu.sync_copy(rows_a, h_hbm.at[c, q, pl.ds(row0, CB), :])
            return carry

        lax.fori_loop(0, RPT // CB, dump, 0)
        plsc.subcore_barrier()


def kernel(feats_node, feats_edge, edge_index, W_node, W_edge, W_attn):
    src_i = edge_index[0].astype(jnp.int32)
    dst_i = edge_index[1].astype(jnp.int32)
    packed = jnp.left_shift(dst_i, PKS) + src_i

    BN = 1000
    z00, z01, z10, z11, s12 = pl.pallas_call(
        _tc_head_body,
        grid=(N // BN,),
        in_specs=[
            pl.BlockSpec((BN, 128), lambda i: (i, 0)),
            pl.BlockSpec((128, 128), lambda i: (0, 0)),
            pl.BlockSpec((1, 384), lambda i: (0, 0)),
        ],
        out_specs=[
            pl.BlockSpec((BN, 32), lambda i: (i, 0)),
            pl.BlockSpec((BN, 32), lambda i: (i, 0)),
            pl.BlockSpec((BN, 32), lambda i: (i, 0)),
            pl.BlockSpec((BN, 32), lambda i: (i, 0)),
            pl.BlockSpec((BN, 2), lambda i: (i, 0)),
        ],
        out_shape=[
            jax.ShapeDtypeStruct((N, 32), jnp.float32),
            jax.ShapeDtypeStruct((N, 32), jnp.float32),
            jax.ShapeDtypeStruct((N, 32), jnp.float32),
            jax.ShapeDtypeStruct((N, 32), jnp.float32),
            jax.ShapeDtypeStruct((N, 2), jnp.float32),
        ],
    )(feats_node, W_node, W_attn)

    BE = 3200
    se = pl.pallas_call(
        _tc_se_body,
        grid=(E // BE,),
        in_specs=[
            pl.BlockSpec((BE, 16), lambda i: (i, 0)),
            pl.BlockSpec((128, 16), lambda i: (0, 0)),
            pl.BlockSpec((1, 384), lambda i: (0, 0)),
        ],
        out_specs=pl.BlockSpec((BE, 1), lambda i: (i, 0)),
        out_shape=jax.ShapeDtypeStruct((E, 1), jnp.float32),
    )(feats_edge, W_edge, W_attn).reshape(NS, EPT)
    s12t = s12.T
    pk2 = packed.reshape(NS, NCH, CB)

    mesh = plsc.VectorSubcoreMesh(core_axis_name="c", subcore_axis_name="s")

    h2 = pl.kernel(
        _sc_body,
        out_type=jax.ShapeDtypeStruct((NC, 2, N, DQ), jnp.float32),
        mesh=mesh,
        compiler_params=pltpu.CompilerParams(needs_layout_passes=False,
                                             use_tc_tiling_on_sc=False),
        scratch_types=[
            pltpu.VMEM((2 * N,), jnp.float32),
            pltpu.VMEM((NCH, CB), jnp.int32),
            pltpu.VMEM((NCH, CB), jnp.int32),
            pltpu.VMEM((EPT,), jnp.float32),
            pltpu.VMEM((NP,), jnp.float32),
            pltpu.VMEM((NP,), jnp.float32),
            pltpu.VMEM((CB, DQ), jnp.float32),
            pltpu.VMEM((CB, DQ), jnp.float32),
            pltpu.VMEM((CB,), jnp.float32),
            pltpu.VMEM((SR,), jnp.float32),
            pltpu.VMEM((SR,), jnp.float32),
            pltpu.VMEM_SHARED((NS, NPQ), jnp.float32),
            pltpu.VMEM_SHARED((N, DQ), jnp.float32),
            pltpu.VMEM_SHARED((N,), jnp.float32),
            pltpu.SemaphoreType.DMA,
            pltpu.SemaphoreType.DMA,
            pltpu.SemaphoreType.DMA,
            pltpu.SemaphoreType.DMA,
            pltpu.SemaphoreType.DMA,
            pltpu.SemaphoreType.DMA,
        ],
    )(pk2, s12t, se, z00, z01, z10, z11)
    return jnp.moveaxis(h2.reshape(4, N, DQ), 0, 1).reshape(N, D)
